# baseline (speedup 1.0000x reference)
import functools

import jax
import jax.numpy as jnp
from jax import lax
from jax.experimental import pallas as pl
from jax.experimental.pallas import tpu as pltpu
from jax.experimental.pallas import tpu_sc as plsc

_D = 2048
_B = 4 * 8192
_NC = 2
_NS = 16
_NW = _NC * _NS
_BPW = _B // _NW
_C = 8
_NBUF = 7
_P = 3
_NCHUNK = _BPW // _C


def _embed_body(idx_hbm, table_hbm, out_hbm, idx_v, rows_v, gsem, ssem):
    wid = lax.axis_index("s") * _NC + lax.axis_index("c")
    base = wid * _BPW
    pltpu.sync_copy(idx_hbm.at[pl.ds(base, _BPW)], idx_v)

    def gather(c, b):
        pltpu.async_copy(
            table_hbm.at[idx_v.at[pl.ds(c * _C, _C)]], rows_v.at[b], gsem
        )

    def gwait(b):
        pltpu.make_async_copy(
            table_hbm.at[idx_v.at[pl.ds(0, _C)]], rows_v.at[b], gsem
        ).wait()

    def scatter(c, b):
        pltpu.async_copy(
            rows_v.at[b], out_hbm.at[pl.ds(base + c * _C, _C)], ssem
        )

    def swait(b):
        pltpu.make_async_copy(
            rows_v.at[b], out_hbm.at[pl.ds(base, _C)], ssem
        ).wait()

    for c in range(_P):
        gather(c, c)

    n_outer = (_NCHUNK + _NBUF - 1) // _NBUF

    def step(i, carry):
        c0 = i * _NBUF
        for b in range(_NBUF):
            c = c0 + b

            @pl.when(c < _NCHUNK)
            def _():
                gwait(b % _NBUF)
                scatter(c, b % _NBUF)

            @pl.when(jnp.logical_and(c >= _P, c < _NCHUNK + _P))
            def _():
                swait((c - _P) % _NBUF)

            @pl.when(c + _P < _NCHUNK)
            def _():
                gather(c + _P, (c + _P) % _NBUF)
        return carry

    lax.fori_loop(0, n_outer, step, 0)


@functools.partial(
    pl.kernel,
    mesh=plsc.VectorSubcoreMesh(core_axis_name="c", subcore_axis_name="s"),
    out_type=jax.ShapeDtypeStruct((_B, _D), jnp.float32),
    scratch_types=[
        pltpu.VMEM((_BPW,), jnp.int32),
        pltpu.VMEM((_NBUF, _C, _D), jnp.float32),
        pltpu.SemaphoreType.DMA,
        pltpu.SemaphoreType.DMA,
    ],
)
def _embed(idx_hbm, table_hbm, out_hbm, idx_v, rows_v, gsem, ssem):
    _embed_body(idx_hbm, table_hbm, out_hbm, idx_v, rows_v, gsem, ssem)


def kernel(input_ids, embed_table):
    ids = input_ids.reshape(-1).astype(jnp.int32)
    out = _embed(ids, embed_table)
    return out.reshape(input_ids.shape + (embed_table.shape[1],))


# speedup vs baseline: 1.0018x; 1.0018x over previous
import functools

import jax
import jax.numpy as jnp
from jax import lax
from jax.experimental import pallas as pl
from jax.experimental.pallas import tpu as pltpu
from jax.experimental.pallas import tpu_sc as plsc

_D = 2048
_B = 4 * 8192
_NC = 2
_NS = 16
_NW = _NC * _NS
_BPW = _B // _NW
_C = 8
_NBUF = 6
_P = 3
_NCHUNK = _BPW // _C


def _embed_body(idx_hbm, table_hbm, out_hbm, idx_v, rows_v, gsem, ssem):
    wid = lax.axis_index("s") * _NC + lax.axis_index("c")
    base = wid * _BPW
    pltpu.sync_copy(idx_hbm.at[pl.ds(base, _BPW)], idx_v)

    def gather(c, b):
        pltpu.async_copy(
            table_hbm.at[idx_v.at[pl.ds(c * _C, _C)]], rows_v.at[b], gsem
        )

    def gwait(b):
        pltpu.make_async_copy(
            table_hbm.at[idx_v.at[pl.ds(0, _C)]], rows_v.at[b], gsem
        ).wait()

    def scatter(c, b):
        pltpu.async_copy(
            rows_v.at[b], out_hbm.at[pl.ds(base + c * _C, _C)], ssem
        )

    def swait(b):
        pltpu.make_async_copy(
            rows_v.at[b], out_hbm.at[pl.ds(base, _C)], ssem
        ).wait()

    for c in range(_P):
        gather(c, c)

    n_outer = (_NCHUNK + _NBUF - 1) // _NBUF

    def step(i, carry):
        c0 = i * _NBUF
        for b in range(_NBUF):
            c = c0 + b

            @pl.when(jnp.logical_and(c >= _P, c < _NCHUNK + _P))
            def _():
                swait((c - _P) % _NBUF)

            @pl.when(c + _P < _NCHUNK)
            def _():
                gather(c + _P, (c + _P) % _NBUF)

            @pl.when(c < _NCHUNK)
            def _():
                gwait(b)
                scatter(c, b)
        return carry

    lax.fori_loop(0, n_outer, step, 0)


@functools.partial(
    pl.kernel,
    mesh=plsc.VectorSubcoreMesh(core_axis_name="c", subcore_axis_name="s"),
    out_type=jax.ShapeDtypeStruct((_B, _D), jnp.float32),
    scratch_types=[
        pltpu.VMEM((_BPW,), jnp.int32),
        pltpu.VMEM((_NBUF, _C, _D), jnp.float32),
        pltpu.SemaphoreType.DMA,
        pltpu.SemaphoreType.DMA,
    ],
)
def _embed(idx_hbm, table_hbm, out_hbm, idx_v, rows_v, gsem, ssem):
    _embed_body(idx_hbm, table_hbm, out_hbm, idx_v, rows_v, gsem, ssem)


def kernel(input_ids, embed_table):
    ids = input_ids.reshape(-1).astype(jnp.int32)
    out = _embed(ids, embed_table)
    return out.reshape(input_ids.shape + (embed_table.shape[1],))
